# baseline (speedup 1.0000x reference)
import functools

import jax
import jax.numpy as jnp
from jax import lax
from jax.experimental import pallas as pl
from jax.experimental.pallas import tpu as pltpu
from jax.experimental.pallas import tpu_sc as plsc

EMB = 32
NW = 32
GRAN = 128
NBUF = 8
S = 50
B = 16384
NG = S * (B // GRAN)
G_PER_W = NG // NW
JJ = B // GRAN
N_SUPER = G_PER_W // NBUF


def _make_kernel():
    mesh = plsc.VectorSubcoreMesh(core_axis_name="c", subcore_axis_name="s")

    @functools.partial(
        pl.kernel,
        mesh=mesh,
        out_type=jax.ShapeDtypeStruct((S, EMB // 8, JJ, 8 * GRAN), jnp.float32),
        scratch_types=[
            pltpu.VMEM((G_PER_W, GRAN), jnp.int32),
            [pltpu.VMEM((GRAN, EMB), jnp.float32) for _ in range(NBUF)],
            [pltpu.VMEM((EMB // 8, 8 * GRAN), jnp.float32) for _ in range(2)],
            [pltpu.SemaphoreType.DMA for _ in range(NBUF)],
            [pltpu.SemaphoreType.DMA for _ in range(2)],
        ],
        compiler_params=pltpu.CompilerParams(
            use_tc_tiling_on_sc=False, needs_layout_passes=False
        ),
    )
    def k(x_hbm, table_hbm, z_hbm, idx_v, gbufs, zbufs, gsems, zsems):
        wid = lax.axis_index("s") * 2 + lax.axis_index("c")
        g0 = wid * G_PER_W
        pltpu.sync_copy(x_hbm.at[pl.ds(g0, G_PER_W)], idx_v)

        iota16 = lax.iota(jnp.int32, 16)
        cvecs = [cb * 16 + iota16 for cb in range(8)]

        def fire(b, t):
            pltpu.async_copy(
                table_hbm.at[idx_v.at[t]],
                gbufs[b],
                gsems[b],
            )

        for b in range(NBUF):
            fire(b, b)

        def body(u, carry):
            for b in range(NBUF):
                t = u * NBUF + b
                p = b % 2
                g = g0 + t
                s = g // JJ
                jj = lax.rem(g, JJ)
                pltpu.make_async_copy(
                    table_hbm.at[pl.ds(0, GRAN)],
                    gbufs[b],
                    gsems[b],
                ).wait()

                def zwait():
                    pltpu.make_async_copy(
                        zbufs[p], z_hbm.at[0, :, 0], zsems[p]
                    ).wait()

                if b < 2:

                    @pl.when(u >= 1)
                    def _():
                        zwait()

                else:
                    zwait()

                def transpose_k(kk, c):
                    for db in range(2):
                        dvec = db * 16 + ((iota16 + kk) & 15)
                        rowvec = lax.shift_right_logical(dvec, 3)
                        offpart = lax.shift_left(dvec & 7, 7)
                        for cb in range(8):
                            val = plsc.load_gather(
                                gbufs[b], [cvecs[cb], dvec]
                            )
                            plsc.store_scatter(
                                zbufs[p],
                                [rowvec, offpart + cvecs[cb]],
                                val,
                            )
                    return c

                lax.fori_loop(0, 16, transpose_k, 0)

                pltpu.async_copy(zbufs[p], z_hbm.at[s, :, jj], zsems[p])

                @pl.when(u < N_SUPER - 1)
                def _():
                    fire(b, t + NBUF)

            return carry

        lax.fori_loop(0, N_SUPER, body, 0)
        for p in range(2):
            pltpu.make_async_copy(zbufs[p], z_hbm.at[0, :, 0], zsems[p]).wait()

    return k


@jax.jit
def kernel(x, table):
    x4 = x.T.reshape(NG, GRAN).astype(jnp.int32)
    z = _make_kernel()(x4, table)
    return (
        z.reshape(S, EMB // 8, JJ, 8, GRAN)
        .transpose(2, 4, 0, 1, 3)
        .reshape(B, S, EMB)
    )


# speedup vs baseline: 1.0081x; 1.0081x over previous
import functools

import jax
import jax.numpy as jnp
from jax import lax
from jax.experimental import pallas as pl
from jax.experimental.pallas import tpu as pltpu
from jax.experimental.pallas import tpu_sc as plsc

EMB = 32
NW = 32
GRAN = 128
NBUF = 4
S = 50
B = 16384
NG = S * (B // GRAN)
G_PER_W = NG // NW
JJ = B // GRAN
N_SUPER = G_PER_W // NBUF


def _make_kernel():
    mesh = plsc.VectorSubcoreMesh(core_axis_name="c", subcore_axis_name="s")

    @functools.partial(
        pl.kernel,
        mesh=mesh,
        out_type=jax.ShapeDtypeStruct((S, EMB // 8, JJ, 8 * GRAN), jnp.float32),
        scratch_types=[
            pltpu.VMEM((G_PER_W, GRAN), jnp.int32),
            [pltpu.VMEM((GRAN, EMB), jnp.float32) for _ in range(NBUF)],
            [pltpu.VMEM((EMB // 8, 8 * GRAN), jnp.float32) for _ in range(2)],
            [pltpu.SemaphoreType.DMA for _ in range(NBUF)],
            [pltpu.SemaphoreType.DMA for _ in range(2)],
        ],
        compiler_params=pltpu.CompilerParams(
            use_tc_tiling_on_sc=False, needs_layout_passes=False
        ),
    )
    def k(x_hbm, table_hbm, z_hbm, idx_v, gbufs, zbufs, gsems, zsems):
        wid = lax.axis_index("s") * 2 + lax.axis_index("c")
        g0 = wid * G_PER_W
        pltpu.sync_copy(x_hbm.at[pl.ds(g0, G_PER_W)], idx_v)

        iota16 = lax.iota(jnp.int32, 16)
        cvecs = [cb * 16 + iota16 for cb in range(8)]

        def fire(b, t):
            pltpu.async_copy(
                table_hbm.at[idx_v.at[t]],
                gbufs[b],
                gsems[b],
            )

        for b in range(NBUF):
            fire(b, b)

        def body(u, carry):
            for b in range(NBUF):
                t = u * NBUF + b
                p = b % 2
                g = g0 + t
                s = g // JJ
                jj = lax.rem(g, JJ)
                pltpu.make_async_copy(
                    table_hbm.at[pl.ds(0, GRAN)],
                    gbufs[b],
                    gsems[b],
                ).wait()

                def zwait():
                    pltpu.make_async_copy(
                        zbufs[p], z_hbm.at[0, :, 0], zsems[p]
                    ).wait()

                if b < 2:

                    @pl.when(u >= 1)
                    def _():
                        zwait()

                else:
                    zwait()

                def transpose_k(kk, c):
                    for db in range(2):
                        dvec = db * 16 + ((iota16 + kk) & 15)
                        rowvec = lax.shift_right_logical(dvec, 3)
                        offpart = lax.shift_left(dvec & 7, 7)
                        for cb in range(8):
                            val = plsc.load_gather(
                                gbufs[b], [cvecs[cb], dvec]
                            )
                            plsc.store_scatter(
                                zbufs[p],
                                [rowvec, offpart + cvecs[cb]],
                                val,
                            )
                    return c

                lax.fori_loop(0, 16, transpose_k, 0)

                pltpu.async_copy(zbufs[p], z_hbm.at[s, :, jj], zsems[p])

                @pl.when(u < N_SUPER - 1)
                def _():
                    fire(b, t + NBUF)

            return carry

        lax.fori_loop(0, N_SUPER, body, 0)
        for p in range(2):
            pltpu.make_async_copy(zbufs[p], z_hbm.at[0, :, 0], zsems[p]).wait()

    return k


@jax.jit
def kernel(x, table):
    x4 = x.T.reshape(NG, GRAN).astype(jnp.int32)
    z = _make_kernel()(x4, table)
    return (
        z.reshape(S, EMB // 8, JJ, 8, GRAN)
        .transpose(2, 4, 0, 1, 3)
        .reshape(B, S, EMB)
    )
